# baseline (speedup 1.0000x reference)
import jax
import jax.numpy as jnp
from jax import lax
from jax.experimental import pallas as pl
from jax.experimental.pallas import tpu as pltpu
from jax.experimental.pallas import tpu_sc as plsc

_BM = 8192
_BN = 8192

_NW = 32
_SUB = 128
_NSUB = 25


def _mlp_body(x_ref, wn1, bn1, wn2, bn2, wr1, br1, wr2, br2, ws1, bs1, ws2,
              bs2, out_ref):
    x = x_ref[0]
    x6 = x[0:8].astype(jnp.bfloat16)
    xall = x.astype(jnp.bfloat16)

    def layer(w, b, xin):
        h = lax.dot_general(w[...], xin, (((1,), (0,)), ((), ())),
                            preferred_element_type=jnp.float32)
        return jnp.maximum(h + b[...], 0.0)

    hn = layer(wn1, bn1, x6)
    hr = layer(wr1, br1, x6)
    hs = layer(ws1, bs1, xall)
    fn = layer(wn2, bn2, hn.astype(jnp.bfloat16))
    fr = layer(wr2, br2, hr.astype(jnp.bfloat16))
    fs = layer(ws2, bs2, hs.astype(jnp.bfloat16))
    g = fn + fr + fs
    gt = g.T.astype(jnp.bfloat16)
    out_ref[...] = pltpu.bitcast(gt, jnp.float32)


def _merge_body(feat_ref, c_ref, idx_ref, out_ref):
    ct = c_ref[...].T
    w = lax.bitcast_convert_type(ct, jnp.int32)
    even = lax.bitcast_convert_type(w << 16, jnp.float32)
    odd = lax.bitcast_convert_type(w & jnp.int32(-65536), jnp.float32)
    idxb = idx_ref[0]
    val = jnp.where((idxb & 1) == 1, odd, even)
    out_ref[0] = feat_ref[0] + jnp.where(idxb >= 0, val, 0.0)


def _sc_gather_body(nrows, g_hbm, idxp_hbm, out_hbm, idx_v, buf0, buf1,
                    sem0, sem1):
    wid = lax.axis_index("s") * 2 + lax.axis_index("c")
    rb = jnp.minimum(wid * _NSUB, nrows - _NSUB)
    pltpu.sync_copy(idxp_hbm.at[wid], idx_v)
    bufs = (buf0, buf1)
    sems = (sem0, sem1)
    d = pltpu.async_copy(g_hbm.at[idx_v.at[0]], buf0, sem0)
    for j in range(_NSUB):
        s = j & 1
        d.wait()
        if j + 1 < _NSUB:
            d = pltpu.async_copy(g_hbm.at[idx_v.at[j + 1]], bufs[1 - s],
                                 sems[1 - s])
        pltpu.sync_copy(bufs[s], out_hbm.at[pl.ds((rb + j) * _SUB, _SUB)])


def kernel(feat, mv_feat, pc2pc_idx, Wn1, bn1, Wn2, bn2, Wr1, br1, Wr2, br2,
           Ws1, bs1, Ws2, bs2):
    B, C, N = feat.shape
    M = mv_feat.shape[2]
    Cin = mv_feat.shape[1]

    wn1p = jnp.pad(Wn1, ((0, 0), (0, 5))).astype(jnp.bfloat16)
    wr1p = jnp.pad(Wr1, ((0, 0), (3, 2))).astype(jnp.bfloat16)
    ws1p = jnp.pad(Ws1, ((0, 0), (6, 0))).astype(jnp.bfloat16)
    wn2b = Wn2.astype(jnp.bfloat16)
    wr2b = Wr2.astype(jnp.bfloat16)
    ws2b = Ws2.astype(jnp.bfloat16)
    b2d = lambda b: b.reshape(C, 1)
    weights = (wn1p, b2d(bn1), wn2b, b2d(bn2), wr1p, b2d(br1), wr2b,
               b2d(br2), ws1p, b2d(bs1), ws2b, b2d(bs2))

    nmb = pl.cdiv(M, _BM)
    wspec = lambda shape: pl.BlockSpec(shape, lambda i: (0, 0))

    def mlp_call(b):
        return pl.pallas_call(
            _mlp_body,
            grid=(nmb,),
            in_specs=[
                pl.BlockSpec((1, Cin, _BM), lambda i, b=b: (b, 0, i)),
                wspec((C, 8)), wspec((C, 1)), wspec((C, C)), wspec((C, 1)),
                wspec((C, 8)), wspec((C, 1)), wspec((C, C)), wspec((C, 1)),
                wspec((C, Cin)), wspec((C, 1)), wspec((C, C)), wspec((C, 1)),
            ],
            out_specs=pl.BlockSpec((_BM // 2, C), lambda i: (i, 0)),
            out_shape=jax.ShapeDtypeStruct((M // 2, C), jnp.float32),
            compiler_params=pltpu.CompilerParams(
                dimension_semantics=("parallel",)),
        )(mv_feat, *weights)

    g0 = mlp_call(0)
    g1 = mlp_call(1)

    nrows = pl.cdiv(N, _SUB)
    npad = nrows * _SUB
    idx = pc2pc_idx.reshape(B, N).astype(jnp.int32)
    idx_safe = jnp.where(idx >= 0, idx, 0) >> 1
    idxp = jnp.pad(idx_safe, ((0, 0), (0, npad - N))).reshape(
        B, nrows, _SUB)
    head = idxp[:, :(_NW - 1) * _NSUB].reshape(B, _NW - 1, _NSUB, _SUB)
    tail = idxp[:, nrows - _NSUB:].reshape(B, 1, _NSUB, _SUB)
    idxw = jnp.concatenate([head, tail], axis=1)

    def sc_gather(g, idxw_b):
        return pl.kernel(
            lambda *a: _sc_gather_body(nrows, *a),
            out_type=jax.ShapeDtypeStruct((npad, C), jnp.float32),
            mesh=plsc.VectorSubcoreMesh(core_axis_name="c",
                                        subcore_axis_name="s"),
            scratch_types=[
                pltpu.VMEM((_NSUB, _SUB), jnp.int32),
                pltpu.VMEM((_SUB, C), jnp.float32),
                pltpu.VMEM((_SUB, C), jnp.float32),
                pltpu.SemaphoreType.DMA,
                pltpu.SemaphoreType.DMA,
            ],
        )(g, idxw_b)

    c0 = sc_gather(g0, idxw[0])
    c1 = sc_gather(g1, idxw[1])
    idx3 = idx.reshape(B, 1, N)

    nnb = pl.cdiv(N, _BN)

    def merge_call(b, c):
        return pl.pallas_call(
            _merge_body,
            grid=(nnb,),
            in_specs=[
                pl.BlockSpec((1, C, _BN), lambda i, b=b: (b, 0, i)),
                pl.BlockSpec((_BN, C), lambda i: (i, 0)),
                pl.BlockSpec((1, 1, _BN), lambda i, b=b: (b, 0, i)),
            ],
            out_specs=pl.BlockSpec((1, C, _BN), lambda i: (0, 0, i)),
            out_shape=jax.ShapeDtypeStruct((1, C, N), jnp.float32),
            compiler_params=pltpu.CompilerParams(
                dimension_semantics=("parallel",)),
        )(feat, c, idx3)

    o0 = merge_call(0, c0)
    o1 = merge_call(1, c1)
    return jnp.concatenate([o0, o1], axis=0)


# speedup vs baseline: 1.1008x; 1.1008x over previous
import jax
import jax.numpy as jnp
from jax import lax
from jax.experimental import pallas as pl
from jax.experimental.pallas import tpu as pltpu
from jax.experimental.pallas import tpu_sc as plsc

_BM = 8192
_BN = 8192

_NW = 32
_SUB = 128
_NSUB = 25


def _mlp_body(x_ref, wn1, bn1, wn2, bn2, wr1, br1, wr2, br2, ws1, bs1, ws2,
              bs2, out_ref):
    x = x_ref[0]
    x6 = x[0:8].astype(jnp.bfloat16)
    xall = x.astype(jnp.bfloat16)

    def layer(w, b, xin):
        h = lax.dot_general(w[...], xin, (((1,), (0,)), ((), ())),
                            preferred_element_type=jnp.float32)
        return jnp.maximum(h + b[...], 0.0)

    hn = layer(wn1, bn1, x6)
    hr = layer(wr1, br1, x6)
    hs = layer(ws1, bs1, xall)
    fn = layer(wn2, bn2, hn.astype(jnp.bfloat16))
    fr = layer(wr2, br2, hr.astype(jnp.bfloat16))
    fs = layer(ws2, bs2, hs.astype(jnp.bfloat16))
    g = fn + fr + fs
    gt = g.T.astype(jnp.bfloat16)
    out_ref[...] = pltpu.bitcast(gt, jnp.float32)


def _merge_body(feat_ref, c0_ref, c1_ref, idx_ref, out_ref):
    for b, cr in ((0, c0_ref), (1, c1_ref)):
        ct = cr[...].T
        w = lax.bitcast_convert_type(ct, jnp.int32)
        even = lax.bitcast_convert_type(w << 16, jnp.float32)
        odd = lax.bitcast_convert_type(
            w & jnp.int32(-65536), jnp.float32)
        idxb = idx_ref[b]
        val = jnp.where((idxb & 1) == 1, odd, even)
        out_ref[b] = feat_ref[b] + jnp.where(idxb >= 0, val, 0.0)


def _sc_gather_body(nrows, g_hbm, idxp_hbm, out_hbm, idx_v, buf0, buf1,
                    sem0, sem1):
    wid = lax.axis_index("s") * 2 + lax.axis_index("c")
    rb = jnp.minimum(wid * _NSUB, nrows - _NSUB)
    pltpu.sync_copy(idxp_hbm.at[wid], idx_v)
    bufs = (buf0, buf1)
    sems = (sem0, sem1)
    d = pltpu.async_copy(g_hbm.at[idx_v.at[0]], buf0, sem0)
    for j in range(_NSUB):
        s = j & 1
        d.wait()
        if j + 1 < _NSUB:
            d = pltpu.async_copy(g_hbm.at[idx_v.at[j + 1]], bufs[1 - s],
                                 sems[1 - s])
        pltpu.sync_copy(bufs[s], out_hbm.at[pl.ds((rb + j) * _SUB, _SUB)])


def kernel(feat, mv_feat, pc2pc_idx, Wn1, bn1, Wn2, bn2, Wr1, br1, Wr2, br2,
           Ws1, bs1, Ws2, bs2):
    B, C, N = feat.shape
    M = mv_feat.shape[2]
    Cin = mv_feat.shape[1]

    wn1p = jnp.pad(Wn1, ((0, 0), (0, 5))).astype(jnp.bfloat16)
    wr1p = jnp.pad(Wr1, ((0, 0), (3, 2))).astype(jnp.bfloat16)
    ws1p = jnp.pad(Ws1, ((0, 0), (6, 0))).astype(jnp.bfloat16)
    wn2b = Wn2.astype(jnp.bfloat16)
    wr2b = Wr2.astype(jnp.bfloat16)
    ws2b = Ws2.astype(jnp.bfloat16)
    b2d = lambda b: b.reshape(C, 1)
    weights = (wn1p, b2d(bn1), wn2b, b2d(bn2), wr1p, b2d(br1), wr2b,
               b2d(br2), ws1p, b2d(bs1), ws2b, b2d(bs2))

    nmb = pl.cdiv(M, _BM)
    wspec = lambda shape: pl.BlockSpec(shape, lambda i: (0, 0))

    def mlp_call(b):
        return pl.pallas_call(
            _mlp_body,
            grid=(nmb,),
            in_specs=[
                pl.BlockSpec((1, Cin, _BM), lambda i, b=b: (b, 0, i)),
                wspec((C, 8)), wspec((C, 1)), wspec((C, C)), wspec((C, 1)),
                wspec((C, 8)), wspec((C, 1)), wspec((C, C)), wspec((C, 1)),
                wspec((C, Cin)), wspec((C, 1)), wspec((C, C)), wspec((C, 1)),
            ],
            out_specs=pl.BlockSpec((_BM // 2, C), lambda i: (i, 0)),
            out_shape=jax.ShapeDtypeStruct((M // 2, C), jnp.float32),
            compiler_params=pltpu.CompilerParams(
                dimension_semantics=("parallel",)),
        )(mv_feat, *weights)

    g0 = mlp_call(0)
    g1 = mlp_call(1)

    nrows = pl.cdiv(N, _SUB)
    npad = nrows * _SUB
    idx = pc2pc_idx.reshape(B, N).astype(jnp.int32)
    idx_safe = jnp.where(idx >= 0, idx, 0) >> 1
    idxp = jnp.pad(idx_safe, ((0, 0), (0, npad - N))).reshape(
        B, nrows, _SUB)
    head = idxp[:, :(_NW - 1) * _NSUB].reshape(B, _NW - 1, _NSUB, _SUB)
    tail = idxp[:, nrows - _NSUB:].reshape(B, 1, _NSUB, _SUB)
    idxw = jnp.concatenate([head, tail], axis=1)

    def sc_gather(g, idxw_b):
        return pl.kernel(
            lambda *a: _sc_gather_body(nrows, *a),
            out_type=jax.ShapeDtypeStruct((npad, C), jnp.float32),
            mesh=plsc.VectorSubcoreMesh(core_axis_name="c",
                                        subcore_axis_name="s"),
            scratch_types=[
                pltpu.VMEM((_NSUB, _SUB), jnp.int32),
                pltpu.VMEM((_SUB, C), jnp.float32),
                pltpu.VMEM((_SUB, C), jnp.float32),
                pltpu.SemaphoreType.DMA,
                pltpu.SemaphoreType.DMA,
            ],
        )(g, idxw_b)

    c0 = sc_gather(g0, idxw[0])
    c1 = sc_gather(g1, idxw[1])
    idx3 = idx.reshape(B, 1, N)

    nnb = pl.cdiv(N, _BN)
    merge_call = pl.pallas_call(
        _merge_body,
        grid=(nnb,),
        in_specs=[
            pl.BlockSpec((B, C, _BN), lambda i: (0, 0, i)),
            pl.BlockSpec((_BN, C), lambda i: (i, 0)),
            pl.BlockSpec((_BN, C), lambda i: (i, 0)),
            pl.BlockSpec((B, 1, _BN), lambda i: (0, 0, i)),
        ],
        out_specs=pl.BlockSpec((B, C, _BN), lambda i: (0, 0, i)),
        out_shape=jax.ShapeDtypeStruct((B, C, N), jnp.float32),
        compiler_params=pltpu.CompilerParams(
            dimension_semantics=("parallel",)),
    )
    return merge_call(feat, c0, c1, idx3)
